# Initial kernel scaffold; baseline (speedup 1.0000x reference)
import functools

import jax
import jax.numpy as jnp
from jax import lax
from jax.experimental import pallas as pl
from jax.experimental.pallas import tpu as pltpu
from jax.experimental.pallas import tpu_sc as plsc

NC = 2
NS = 16
LANES = 16
CHUNK = 128
DEG_W = 16
EPS = 1e-5

_MESH = plsc.VectorSubcoreMesh(core_axis_name="c", subcore_axis_name="s")
_PREC = lax.Precision.HIGHEST


def _dot(a, b):
    return jnp.dot(a, b, precision=_PREC, preferred_element_type=jnp.float32)


def _sc_degree(dst):
    E = dst.shape[0]
    n = 10000
    ne_core = E // NC
    ne_w = ne_core // NS
    nfull = ne_w // CHUNK
    tail = ne_w % CHUNK
    rows_sub = n // NS

    @functools.partial(
        pl.kernel,
        out_type=jax.ShapeDtypeStruct((NC * n, DEG_W), jnp.float32),
        mesh=_MESH,
        scratch_types=[
            pltpu.VMEM((CHUNK,), jnp.int32),
            pltpu.VMEM((LANES,), jnp.int32),
            pltpu.VMEM((CHUNK, DEG_W), jnp.float32),
            pltpu.VMEM((CHUNK, DEG_W), jnp.float32),
            pltpu.VMEM_SHARED((n, DEG_W), jnp.float32),
        ],
    )
    def deg_kernel(dst_hbm, out_hbm, idx_v, idxt_v, ones_v, zeros_v, acc_sh):
        cid = lax.axis_index("c")
        sid = lax.axis_index("s")

        @pl.loop(0, CHUNK)
        def _(r):
            ones_v[r, :] = jnp.full((LANES,), 1.0, jnp.float32)
            zeros_v[r, :] = jnp.zeros((LANES,), jnp.float32)

        r0 = sid * rows_sub
        done = 0
        while done < rows_sub:
            step = min(CHUNK, rows_sub - done)
            pltpu.sync_copy(zeros_v.at[pl.ds(0, step)],
                            acc_sh.at[pl.ds(r0 + done, step)])
            done += step
        plsc.subcore_barrier()

        base0 = cid * ne_core + sid * ne_w

        @pl.loop(0, nfull)
        def _(j):
            pltpu.sync_copy(dst_hbm.at[pl.ds(base0 + j * CHUNK, CHUNK)], idx_v)
            pltpu.sync_copy(ones_v, acc_sh.at[idx_v], add=True)

        if tail:
            pltpu.sync_copy(dst_hbm.at[pl.ds(base0 + nfull * CHUNK, tail)], idxt_v)
            pltpu.sync_copy(ones_v.at[pl.ds(0, tail)], acc_sh.at[idxt_v], add=True)

        plsc.subcore_barrier()
        pltpu.sync_copy(acc_sh.at[pl.ds(r0, rows_sub)],
                        out_hbm.at[pl.ds(cid * n + r0, rows_sub)])

    return deg_kernel(dst)


def _sc_aggregate(h, src, dst):
    n, d = h.shape
    E = src.shape[0]
    ne_core = E // NC
    ne_w = ne_core // NS
    nfull = ne_w // CHUNK
    tail = ne_w % CHUNK
    rows_sub = n // NS

    @functools.partial(
        pl.kernel,
        out_type=jax.ShapeDtypeStruct((NC * n, d), jnp.float32),
        mesh=_MESH,
        scratch_types=[
            pltpu.VMEM((CHUNK,), jnp.int32),
            pltpu.VMEM((CHUNK,), jnp.int32),
            pltpu.VMEM((LANES,), jnp.int32),
            pltpu.VMEM((CHUNK, d), jnp.float32),
            pltpu.VMEM_SHARED((n, d), jnp.float32),
            pltpu.SemaphoreType.DMA,
        ],
    )
    def agg_kernel(h_hbm, src_hbm, dst_hbm, out_hbm,
                   srci_v, dsti_v, idxt_v, rows_v, acc_sh, sem):
        cid = lax.axis_index("c")
        sid = lax.axis_index("s")

        @pl.loop(0, CHUNK)
        def _(r):
            for c8 in range(d // LANES):
                rows_v[r, pl.ds(c8 * LANES, LANES)] = jnp.zeros((LANES,), jnp.float32)

        r0 = sid * rows_sub
        done = 0
        while done < rows_sub:
            step = min(CHUNK, rows_sub - done)
            pltpu.sync_copy(rows_v.at[pl.ds(0, step)],
                            acc_sh.at[pl.ds(r0 + done, step)])
            done += step
        plsc.subcore_barrier()

        base0 = cid * ne_core + sid * ne_w

        @pl.loop(0, nfull)
        def _(j):
            b = base0 + j * CHUNK
            pltpu.sync_copy(src_hbm.at[pl.ds(b, CHUNK)], srci_v)
            pltpu.async_copy(h_hbm.at[srci_v], rows_v, sem).wait()
            pltpu.sync_copy(dst_hbm.at[pl.ds(b, CHUNK)], dsti_v)
            pltpu.sync_copy(rows_v, acc_sh.at[dsti_v], add=True)

        if tail:
            b = base0 + nfull * CHUNK
            pltpu.sync_copy(src_hbm.at[pl.ds(b, tail)], idxt_v)
            pltpu.async_copy(h_hbm.at[idxt_v], rows_v.at[pl.ds(0, tail)], sem).wait()
            pltpu.sync_copy(dst_hbm.at[pl.ds(b, tail)], idxt_v)
            pltpu.sync_copy(rows_v.at[pl.ds(0, tail)], acc_sh.at[idxt_v], add=True)

        plsc.subcore_barrier()
        pltpu.sync_copy(acc_sh.at[pl.ds(r0, rows_sub)],
                        out_hbm.at[pl.ds(cid * n + r0, rows_sub)])

    return agg_kernel(h, src, dst)


def _tc_prep(degpart, x, W1):
    n, d = x.shape

    def body(degp_ref, x_ref, w_ref, h1s_ref, dinv2_ref):
        deg = degp_ref[0, :, 0:1] + degp_ref[1, :, 0:1] + 1.0
        dinv = lax.rsqrt(jnp.maximum(deg, 1e-12))
        dinv2 = jnp.broadcast_to(dinv, (n, d))
        dinv2_ref[...] = dinv2
        h1s_ref[...] = _dot(x_ref[...], w_ref[...]) * dinv2

    return pl.pallas_call(
        body,
        out_shape=[jax.ShapeDtypeStruct((n, d), jnp.float32),
                   jax.ShapeDtypeStruct((n, d), jnp.float32)],
    )(degpart, x, W1)


def _combine_bn_relu(acc_ref, hs_ref, dinv2_ref, b_ref, g_ref, bt_ref):
    h = (acc_ref[0] + acc_ref[1] + hs_ref[...]) * dinv2_ref[...] + b_ref[...]
    mu = jnp.mean(h, axis=0)
    var = jnp.mean((h - mu) ** 2, axis=0)
    return jnp.maximum((h - mu) * lax.rsqrt(var + EPS) * g_ref[...] + bt_ref[...],
                       0.0)


def _tc_layer(accpart, hs, dinv2, b, g, bt, Wn):
    n, d = hs.shape

    def body(acc_ref, hs_ref, dinv2_ref, b_ref, g_ref, bt_ref, w_ref, out_ref):
        hn = _combine_bn_relu(acc_ref, hs_ref, dinv2_ref, b_ref, g_ref, bt_ref)
        out_ref[...] = _dot(hn, w_ref[...]) * dinv2_ref[...]

    return pl.pallas_call(
        body,
        out_shape=jax.ShapeDtypeStruct((n, d), jnp.float32),
    )(accpart, hs, dinv2, b, g, bt, Wn)


def _tc_tail(accpart, hs, dinv2, b, g, bt, batch, Wc1p, bc1p, Wc2p, bc2p, We, be):
    n, d = hs.shape
    ngr = 16

    def body(acc_ref, hs_ref, dinv2_ref, b_ref, g_ref, bt_ref, batch_ref,
             wc1_ref, bc1_ref, wc2_ref, bc2_ref, we_ref, be_ref,
             ne_ref, logits_ref, emb_ref):
        hn = _combine_bn_relu(acc_ref, hs_ref, dinv2_ref, b_ref, g_ref, bt_ref)
        ne_ref[...] = hn
        gid = lax.broadcasted_iota(jnp.int32, (ngr, n), 0)
        mask = (batch_ref[...][None, :] == gid).astype(jnp.float32)
        cnt = jnp.maximum(jnp.sum(mask, axis=1), 1.0)
        pooled = _dot(mask, hn) / cnt[:, None]
        z = jnp.maximum(_dot(pooled, wc1_ref[...]) + bc1_ref[...], 0.0)
        logits_ref[...] = _dot(z, wc2_ref[...]) + bc2_ref[...]
        emb_ref[...] = _dot(pooled, we_ref[...]) + be_ref[...]

    return pl.pallas_call(
        body,
        out_shape=[jax.ShapeDtypeStruct((n, d), jnp.float32),
                   jax.ShapeDtypeStruct((ngr, d), jnp.float32),
                   jax.ShapeDtypeStruct((ngr, d), jnp.float32)],
    )(accpart, hs, dinv2, b, g, bt, batch, Wc1p, bc1p, Wc2p, bc2p, We, be)


def kernel(x, edge_index, batch, W1, b1, W2, b2, W3, b3, g1, beta1, g2, beta2,
           g3, beta3, Wc1, bc1, Wc2, bc2, We, be):
    n, d = x.shape
    src = edge_index[0]
    dst = edge_index[1]

    h1 = Wc1.shape[1]
    Wc1p = jnp.zeros((d, d), jnp.float32).at[:, :h1].set(Wc1)
    bc1p = jnp.zeros((d,), jnp.float32).at[:h1].set(bc1)
    Wc2p = jnp.zeros((d, d), jnp.float32).at[:h1, :Wc2.shape[1]].set(Wc2)
    bc2p = jnp.zeros((d,), jnp.float32).at[:Wc2.shape[1]].set(bc2)

    degpart = _sc_degree(dst).reshape(NC, n, DEG_W)
    h1s, dinv2 = _tc_prep(degpart, x, W1)

    acc1 = _sc_aggregate(h1s, src, dst).reshape(NC, n, d)
    h2s = _tc_layer(acc1, h1s, dinv2, b1, g1, beta1, W2)

    acc2 = _sc_aggregate(h2s, src, dst).reshape(NC, n, d)
    h3s = _tc_layer(acc2, h2s, dinv2, b2, g2, beta2, W3)

    acc3 = _sc_aggregate(h3s, src, dst).reshape(NC, n, d)
    node_embeddings, logits_p, embedding = _tc_tail(
        acc3, h3s, dinv2, b3, g3, beta3, batch, Wc1p, bc1p, Wc2p, bc2p, We, be)

    return logits_p[:, :Wc2.shape[1]], embedding, node_embeddings


# speedup vs baseline: 13.4434x; 13.4434x over previous
import functools

import jax
import jax.numpy as jnp
from jax import lax
from jax.experimental import pallas as pl
from jax.experimental.pallas import tpu as pltpu
from jax.experimental.pallas import tpu_sc as plsc

NC = 2
NS = 16
LANES = 16
CHUNK = 128
DEG_W = 128
EPS = 1e-5

_MESH = plsc.VectorSubcoreMesh(core_axis_name="c", subcore_axis_name="s")
_PREC = lax.Precision.HIGHEST


def _dot(a, b):
    return jnp.dot(a, b, precision=_PREC, preferred_element_type=jnp.float32)


def _pad_rows(n):
    return -(-n // (8 * NS)) * (8 * NS)


def _sc_degree(dst, n):
    E = dst.shape[0]
    np_ = _pad_rows(n)
    ne_core = E // NC
    ne_w = ne_core // NS
    nfull = ne_w // CHUNK
    tail = ne_w % CHUNK
    rows_sub = np_ // NS
    w = DEG_W

    @functools.partial(
        pl.kernel,
        out_type=jax.ShapeDtypeStruct((NC * np_, w), jnp.float32),
        mesh=_MESH,
        scratch_types=[
            pltpu.VMEM((CHUNK,), jnp.int32),
            pltpu.VMEM((LANES,), jnp.int32),
            pltpu.VMEM((CHUNK, w), jnp.float32),
            pltpu.VMEM((CHUNK, w), jnp.float32),
            pltpu.VMEM_SHARED((np_, w), jnp.float32),
        ],
    )
    def deg_kernel(dst_hbm, out_hbm, idx_v, idxt_v, ones_v, zeros_v, acc_sh):
        cid = lax.axis_index("c")
        sid = lax.axis_index("s")

        @pl.loop(0, CHUNK)
        def _(r):
            for c8 in range(w // LANES):
                ones_v[r, pl.ds(c8 * LANES, LANES)] = jnp.full((LANES,), 1.0, jnp.float32)
                zeros_v[r, pl.ds(c8 * LANES, LANES)] = jnp.zeros((LANES,), jnp.float32)

        r0 = sid * rows_sub
        done = 0
        while done < rows_sub:
            step = min(CHUNK, rows_sub - done)
            pltpu.sync_copy(zeros_v.at[pl.ds(0, step)],
                            acc_sh.at[pl.ds(r0 + done, step)])
            done += step
        plsc.subcore_barrier()

        base0 = cid * ne_core + sid * ne_w

        @pl.loop(0, nfull)
        def _(j):
            pltpu.sync_copy(dst_hbm.at[pl.ds(base0 + j * CHUNK, CHUNK)], idx_v)
            pltpu.sync_copy(ones_v, acc_sh.at[idx_v], add=True)

        if tail:
            pltpu.sync_copy(dst_hbm.at[pl.ds(base0 + nfull * CHUNK, tail)], idxt_v)
            pltpu.sync_copy(ones_v.at[pl.ds(0, tail)], acc_sh.at[idxt_v], add=True)

        plsc.subcore_barrier()
        pltpu.sync_copy(acc_sh.at[pl.ds(r0, rows_sub)],
                        out_hbm.at[pl.ds(cid * np_ + r0, rows_sub)])

    return deg_kernel(dst)


def _sc_aggregate(h, src, dst):
    n, d = h.shape
    np_ = _pad_rows(n)
    E = src.shape[0]
    ne_core = E // NC
    ne_w = ne_core // NS
    nfull = ne_w // CHUNK
    tail = ne_w % CHUNK
    rows_sub = np_ // NS

    @functools.partial(
        pl.kernel,
        out_type=jax.ShapeDtypeStruct((NC * np_, d), jnp.float32),
        mesh=_MESH,
        scratch_types=[
            pltpu.VMEM((CHUNK,), jnp.int32),
            pltpu.VMEM((CHUNK,), jnp.int32),
            pltpu.VMEM((LANES,), jnp.int32),
            pltpu.VMEM((CHUNK, d), jnp.float32),
            pltpu.VMEM_SHARED((np_, d), jnp.float32),
            pltpu.SemaphoreType.DMA,
        ],
    )
    def agg_kernel(h_hbm, src_hbm, dst_hbm, out_hbm,
                   srci_v, dsti_v, idxt_v, rows_v, acc_sh, sem):
        cid = lax.axis_index("c")
        sid = lax.axis_index("s")

        @pl.loop(0, CHUNK)
        def _(r):
            for c8 in range(d // LANES):
                rows_v[r, pl.ds(c8 * LANES, LANES)] = jnp.zeros((LANES,), jnp.float32)

        r0 = sid * rows_sub
        done = 0
        while done < rows_sub:
            step = min(CHUNK, rows_sub - done)
            pltpu.sync_copy(rows_v.at[pl.ds(0, step)],
                            acc_sh.at[pl.ds(r0 + done, step)])
            done += step
        plsc.subcore_barrier()

        base0 = cid * ne_core + sid * ne_w

        @pl.loop(0, nfull)
        def _(j):
            b = base0 + j * CHUNK
            pltpu.sync_copy(src_hbm.at[pl.ds(b, CHUNK)], srci_v)
            pltpu.async_copy(h_hbm.at[srci_v], rows_v, sem).wait()
            pltpu.sync_copy(dst_hbm.at[pl.ds(b, CHUNK)], dsti_v)
            pltpu.sync_copy(rows_v, acc_sh.at[dsti_v], add=True)

        if tail:
            b = base0 + nfull * CHUNK
            pltpu.sync_copy(src_hbm.at[pl.ds(b, tail)], idxt_v)
            pltpu.async_copy(h_hbm.at[idxt_v], rows_v.at[pl.ds(0, tail)], sem).wait()
            pltpu.sync_copy(dst_hbm.at[pl.ds(b, tail)], idxt_v)
            pltpu.sync_copy(rows_v.at[pl.ds(0, tail)], acc_sh.at[idxt_v], add=True)

        plsc.subcore_barrier()
        pltpu.sync_copy(acc_sh.at[pl.ds(r0, rows_sub)],
                        out_hbm.at[pl.ds(cid * np_ + r0, rows_sub)])

    return agg_kernel(h, src, dst)


def _tc_prep(degpart, x, W1):
    n, d = x.shape

    def body(degp_ref, x_ref, w_ref, h1s_ref, dinv2_ref):
        deg = degp_ref[0, :n, 0:1] + degp_ref[1, :n, 0:1] + 1.0
        dinv = lax.rsqrt(jnp.maximum(deg, 1e-12))
        dinv2 = jnp.broadcast_to(dinv, (n, d))
        dinv2_ref[...] = dinv2
        h1s_ref[...] = _dot(x_ref[...], w_ref[...]) * dinv2

    return pl.pallas_call(
        body,
        out_shape=[jax.ShapeDtypeStruct((n, d), jnp.float32),
                   jax.ShapeDtypeStruct((n, d), jnp.float32)],
    )(degpart, x, W1)


def _combine_bn_relu(acc_ref, hs_ref, dinv2_ref, b_ref, g_ref, bt_ref):
    n = hs_ref.shape[0]
    h = (acc_ref[0, :n] + acc_ref[1, :n] + hs_ref[...]) * dinv2_ref[...] + b_ref[...]
    mu = jnp.mean(h, axis=0)
    var = jnp.mean((h - mu) ** 2, axis=0)
    return jnp.maximum((h - mu) * lax.rsqrt(var + EPS) * g_ref[...] + bt_ref[...],
                       0.0)


def _tc_layer(accpart, hs, dinv2, b, g, bt, Wn):
    n, d = hs.shape

    def body(acc_ref, hs_ref, dinv2_ref, b_ref, g_ref, bt_ref, w_ref, out_ref):
        hn = _combine_bn_relu(acc_ref, hs_ref, dinv2_ref, b_ref, g_ref, bt_ref)
        out_ref[...] = _dot(hn, w_ref[...]) * dinv2_ref[...]

    return pl.pallas_call(
        body,
        out_shape=jax.ShapeDtypeStruct((n, d), jnp.float32),
    )(accpart, hs, dinv2, b, g, bt, Wn)


def _tc_tail(accpart, hs, dinv2, b, g, bt, batch, Wc1p, bc1p, Wc2p, bc2p, We, be):
    n, d = hs.shape
    ngr = 16

    def body(acc_ref, hs_ref, dinv2_ref, b_ref, g_ref, bt_ref, batch_ref,
             wc1_ref, bc1_ref, wc2_ref, bc2_ref, we_ref, be_ref,
             ne_ref, logits_ref, emb_ref):
        hn = _combine_bn_relu(acc_ref, hs_ref, dinv2_ref, b_ref, g_ref, bt_ref)
        ne_ref[...] = hn
        gid = lax.broadcasted_iota(jnp.int32, (ngr, n), 0)
        mask = (batch_ref[...][None, :] == gid).astype(jnp.float32)
        cnt = jnp.maximum(jnp.sum(mask, axis=1), 1.0)
        pooled = _dot(mask, hn) / cnt[:, None]
        z = jnp.maximum(_dot(pooled, wc1_ref[...]) + bc1_ref[...], 0.0)
        logits_ref[...] = _dot(z, wc2_ref[...]) + bc2_ref[...]
        emb_ref[...] = _dot(pooled, we_ref[...]) + be_ref[...]

    return pl.pallas_call(
        body,
        out_shape=[jax.ShapeDtypeStruct((n, d), jnp.float32),
                   jax.ShapeDtypeStruct((ngr, d), jnp.float32),
                   jax.ShapeDtypeStruct((ngr, d), jnp.float32)],
    )(accpart, hs, dinv2, b, g, bt, batch, Wc1p, bc1p, Wc2p, bc2p, We, be)


def kernel(x, edge_index, batch, W1, b1, W2, b2, W3, b3, g1, beta1, g2, beta2,
           g3, beta3, Wc1, bc1, Wc2, bc2, We, be):
    n, d = x.shape
    src = edge_index[0]
    dst = edge_index[1]

    h1 = Wc1.shape[1]
    Wc1p = jnp.zeros((d, d), jnp.float32).at[:, :h1].set(Wc1)
    bc1p = jnp.zeros((d,), jnp.float32).at[:h1].set(bc1)
    Wc2p = jnp.zeros((d, d), jnp.float32).at[:h1, :Wc2.shape[1]].set(Wc2)
    bc2p = jnp.zeros((d,), jnp.float32).at[:Wc2.shape[1]].set(bc2)

    np_ = _pad_rows(n)
    degpart = _sc_degree(dst, n).reshape(NC, np_, DEG_W)
    h1s, dinv2 = _tc_prep(degpart, x, W1)

    acc1 = _sc_aggregate(h1s, src, dst).reshape(NC, np_, d)
    h2s = _tc_layer(acc1, h1s, dinv2, b1, g1, beta1, W2)

    acc2 = _sc_aggregate(h2s, src, dst).reshape(NC, np_, d)
    h3s = _tc_layer(acc2, h2s, dinv2, b2, g2, beta2, W3)

    acc3 = _sc_aggregate(h3s, src, dst).reshape(NC, np_, d)
    node_embeddings, logits_p, embedding = _tc_tail(
        acc3, h3s, dinv2, b3, g3, beta3, batch, Wc1p, bc1p, Wc2p, bc2p, We, be)

    return logits_p[:, :Wc2.shape[1]], embedding, node_embeddings


# speedup vs baseline: 22.6746x; 1.6867x over previous
import functools

import jax
import jax.numpy as jnp
from jax import lax
from jax.experimental import pallas as pl
from jax.experimental.pallas import tpu as pltpu
from jax.experimental.pallas import tpu_sc as plsc

NC = 2
NS = 16
LANES = 16
CHUNK = 64
DEG_W = 128
EPS = 1e-5

_MESH = plsc.VectorSubcoreMesh(core_axis_name="c", subcore_axis_name="s")
_PREC = lax.Precision.HIGHEST


def _dot(a, b):
    return jnp.dot(a, b, precision=_PREC, preferred_element_type=jnp.float32)


def _pad_rows(n):
    return -(-n // (8 * NS)) * (8 * NS)


def _sc_degree(dst, n):
    E = dst.shape[0]
    np_ = _pad_rows(n)
    ne_core = E // NC
    ne_w = ne_core // NS
    nfull = ne_w // CHUNK
    tail = ne_w % CHUNK
    rows_sub = np_ // NS
    w = DEG_W
    KB = 13

    @functools.partial(
        pl.kernel,
        out_type=jax.ShapeDtypeStruct((NC * np_, w), jnp.float32),
        mesh=_MESH,
        scratch_types=[
            pltpu.VMEM((nfull, CHUNK), jnp.int32),
            pltpu.VMEM((LANES,), jnp.int32),
            pltpu.VMEM((CHUNK, w), jnp.float32),
            pltpu.VMEM((CHUNK, w), jnp.float32),
            pltpu.VMEM_SHARED((np_, w), jnp.float32),
            pltpu.SemaphoreType.DMA,
            pltpu.SemaphoreType.DMA,
        ],
    )
    def deg_kernel(dst_hbm, out_hbm, dsti_all, idxt_v, ones_v, zeros_v, acc_sh,
                   isem, ssem):
        cid = lax.axis_index("c")
        sid = lax.axis_index("s")

        @pl.loop(0, CHUNK)
        def _(r):
            for c8 in range(w // LANES):
                ones_v[r, pl.ds(c8 * LANES, LANES)] = jnp.full((LANES,), 1.0, jnp.float32)
                zeros_v[r, pl.ds(c8 * LANES, LANES)] = jnp.zeros((LANES,), jnp.float32)

        base0 = cid * ne_core + sid * ne_w

        @pl.loop(0, nfull)
        def _(j):
            pltpu.async_copy(dst_hbm.at[pl.ds(base0 + j * CHUNK, CHUNK)],
                             dsti_all.at[j], isem)
        if tail:
            pltpu.sync_copy(dst_hbm.at[pl.ds(base0 + nfull * CHUNK, tail)], idxt_v)

        r0 = sid * rows_sub
        done = 0
        while done < rows_sub:
            step = min(CHUNK, rows_sub - done)
            pltpu.sync_copy(zeros_v.at[pl.ds(0, step)],
                            acc_sh.at[pl.ds(r0 + done, step)])
            done += step
        @pl.loop(0, nfull)
        def _(j):
            pltpu.make_async_copy(dst_hbm.at[pl.ds(base0 + j * CHUNK, CHUNK)],
                                  dsti_all.at[j], isem).wait()
        plsc.subcore_barrier()

        assert nfull % KB == 0
        @pl.loop(0, nfull, step=KB)
        def _(b0):
            for jo in range(KB):
                pltpu.async_copy(ones_v, acc_sh.at[dsti_all.at[b0 + jo]], ssem,
                                 add=True)
            for jo in range(KB):
                pltpu.make_async_copy(ones_v, acc_sh.at[dsti_all.at[b0 + jo]],
                                      ssem).wait()
        if tail:
            pltpu.sync_copy(ones_v.at[pl.ds(0, tail)], acc_sh.at[idxt_v], add=True)

        plsc.subcore_barrier()
        pltpu.sync_copy(acc_sh.at[pl.ds(r0, rows_sub)],
                        out_hbm.at[pl.ds(cid * np_ + r0, rows_sub)])

    return deg_kernel(dst)


def _sc_aggregate(h, src, dst):
    n, d = h.shape
    np_ = _pad_rows(n)
    E = src.shape[0]
    ne_core = E // NC
    ne_w = ne_core // NS
    nfull = ne_w // CHUNK
    tail = ne_w % CHUNK
    rows_sub = np_ // NS
    assert nfull >= 4 and nfull % 2 == 0

    @functools.partial(
        pl.kernel,
        out_type=jax.ShapeDtypeStruct((NC * np_, d), jnp.float32),
        mesh=_MESH,
        scratch_types=[
            pltpu.VMEM((ne_w,), jnp.int32),
            pltpu.VMEM((nfull, CHUNK), jnp.int32),
            pltpu.VMEM((LANES,), jnp.int32),
            pltpu.VMEM((CHUNK, d), jnp.float32),
            pltpu.VMEM((CHUNK, d), jnp.float32),
            pltpu.VMEM_SHARED((np_, d), jnp.float32),
            pltpu.SemaphoreType.DMA,
            pltpu.SemaphoreType.DMA,
            pltpu.SemaphoreType.DMA,
        ],
    )
    def agg_kernel(h_hbm, src_hbm, dst_hbm, out_hbm,
                   srci_all, dsti_all, idxt_v, rows_a, rows_b, acc_sh,
                   gsem_a, gsem_b, isem):
        cid = lax.axis_index("c")
        sid = lax.axis_index("s")
        base0 = cid * ne_core + sid * ne_w

        @pl.loop(0, nfull)
        def _(j):
            pltpu.async_copy(dst_hbm.at[pl.ds(base0 + j * CHUNK, CHUNK)],
                             dsti_all.at[j], isem)
        pltpu.sync_copy(src_hbm.at[pl.ds(base0, ne_w)], srci_all)

        @pl.loop(0, CHUNK)
        def _(r):
            for c8 in range(d // LANES):
                rows_a[r, pl.ds(c8 * LANES, LANES)] = jnp.zeros((LANES,), jnp.float32)

        r0 = sid * rows_sub
        done = 0
        while done < rows_sub:
            step = min(CHUNK, rows_sub - done)
            pltpu.sync_copy(rows_a.at[pl.ds(0, step)],
                            acc_sh.at[pl.ds(r0 + done, step)])
            done += step
        @pl.loop(0, nfull)
        def _(j):
            pltpu.make_async_copy(dst_hbm.at[pl.ds(base0 + j * CHUNK, CHUNK)],
                                  dsti_all.at[j], isem).wait()
        plsc.subcore_barrier()

        def gather(j, buf, sem):
            return pltpu.async_copy(
                h_hbm.at[srci_all.at[pl.ds(j * CHUNK, CHUNK)]], buf, sem)

        def gather_wait(j, buf, sem):
            pltpu.make_async_copy(
                h_hbm.at[srci_all.at[pl.ds(j * CHUNK, CHUNK)]], buf, sem).wait()

        def scatter(j, buf):
            pltpu.sync_copy(buf, acc_sh.at[dsti_all.at[j]], add=True)

        gather(0, rows_a, gsem_a)

        @pl.loop(0, nfull - 2, step=2)
        def _(j):
            gather(j + 1, rows_b, gsem_b)
            gather_wait(j, rows_a, gsem_a)
            scatter(j, rows_a)
            gather(j + 2, rows_a, gsem_a)
            gather_wait(j + 1, rows_b, gsem_b)
            scatter(j + 1, rows_b)

        jj = nfull - 2
        gather(jj + 1, rows_b, gsem_b)
        gather_wait(jj, rows_a, gsem_a)
        scatter(jj, rows_a)
        gather_wait(jj + 1, rows_b, gsem_b)
        scatter(jj + 1, rows_b)

        if tail:
            b = base0 + nfull * CHUNK
            pltpu.sync_copy(src_hbm.at[pl.ds(b, tail)], idxt_v)
            pltpu.async_copy(h_hbm.at[idxt_v], rows_a.at[pl.ds(0, tail)],
                             gsem_a).wait()
            pltpu.sync_copy(dst_hbm.at[pl.ds(b, tail)], idxt_v)
            pltpu.sync_copy(rows_a.at[pl.ds(0, tail)], acc_sh.at[idxt_v], add=True)

        plsc.subcore_barrier()
        pltpu.sync_copy(acc_sh.at[pl.ds(r0, rows_sub)],
                        out_hbm.at[pl.ds(cid * np_ + r0, rows_sub)])

    return agg_kernel(h, src, dst)


def _tc_prep(degpart, x, W1):
    n, d = x.shape

    def body(degp_ref, x_ref, w_ref, h1s_ref, dinv2_ref):
        deg = degp_ref[0, :n, 0:1] + degp_ref[1, :n, 0:1] + 1.0
        dinv = lax.rsqrt(jnp.maximum(deg, 1e-12))
        dinv2 = jnp.broadcast_to(dinv, (n, d))
        dinv2_ref[...] = dinv2
        h1s_ref[...] = _dot(x_ref[...], w_ref[...]) * dinv2

    return pl.pallas_call(
        body,
        out_shape=[jax.ShapeDtypeStruct((n, d), jnp.float32),
                   jax.ShapeDtypeStruct((n, d), jnp.float32)],
    )(degpart, x, W1)


def _combine_bn_relu(acc_ref, hs_ref, dinv2_ref, b_ref, g_ref, bt_ref):
    n = hs_ref.shape[0]
    h = (acc_ref[0, :n] + acc_ref[1, :n] + hs_ref[...]) * dinv2_ref[...] + b_ref[...]
    mu = jnp.mean(h, axis=0)
    var = jnp.mean((h - mu) ** 2, axis=0)
    return jnp.maximum((h - mu) * lax.rsqrt(var + EPS) * g_ref[...] + bt_ref[...],
                       0.0)


def _tc_layer(accpart, hs, dinv2, b, g, bt, Wn):
    n, d = hs.shape

    def body(acc_ref, hs_ref, dinv2_ref, b_ref, g_ref, bt_ref, w_ref, out_ref):
        hn = _combine_bn_relu(acc_ref, hs_ref, dinv2_ref, b_ref, g_ref, bt_ref)
        out_ref[...] = _dot(hn, w_ref[...]) * dinv2_ref[...]

    return pl.pallas_call(
        body,
        out_shape=jax.ShapeDtypeStruct((n, d), jnp.float32),
    )(accpart, hs, dinv2, b, g, bt, Wn)


def _tc_tail(accpart, hs, dinv2, b, g, bt, batch, Wc1p, bc1p, Wc2p, bc2p, We, be):
    n, d = hs.shape
    ngr = 16

    def body(acc_ref, hs_ref, dinv2_ref, b_ref, g_ref, bt_ref, batch_ref,
             wc1_ref, bc1_ref, wc2_ref, bc2_ref, we_ref, be_ref,
             ne_ref, logits_ref, emb_ref):
        hn = _combine_bn_relu(acc_ref, hs_ref, dinv2_ref, b_ref, g_ref, bt_ref)
        ne_ref[...] = hn
        gid = lax.broadcasted_iota(jnp.int32, (ngr, n), 0)
        mask = (batch_ref[...][None, :] == gid).astype(jnp.float32)
        cnt = jnp.maximum(jnp.sum(mask, axis=1), 1.0)
        pooled = _dot(mask, hn) / cnt[:, None]
        z = jnp.maximum(_dot(pooled, wc1_ref[...]) + bc1_ref[...], 0.0)
        logits_ref[...] = _dot(z, wc2_ref[...]) + bc2_ref[...]
        emb_ref[...] = _dot(pooled, we_ref[...]) + be_ref[...]

    return pl.pallas_call(
        body,
        out_shape=[jax.ShapeDtypeStruct((n, d), jnp.float32),
                   jax.ShapeDtypeStruct((ngr, d), jnp.float32),
                   jax.ShapeDtypeStruct((ngr, d), jnp.float32)],
    )(accpart, hs, dinv2, b, g, bt, batch, Wc1p, bc1p, Wc2p, bc2p, We, be)


def kernel(x, edge_index, batch, W1, b1, W2, b2, W3, b3, g1, beta1, g2, beta2,
           g3, beta3, Wc1, bc1, Wc2, bc2, We, be):
    n, d = x.shape
    src = edge_index[0]
    dst = edge_index[1]

    h1 = Wc1.shape[1]
    Wc1p = jnp.zeros((d, d), jnp.float32).at[:, :h1].set(Wc1)
    bc1p = jnp.zeros((d,), jnp.float32).at[:h1].set(bc1)
    Wc2p = jnp.zeros((d, d), jnp.float32).at[:h1, :Wc2.shape[1]].set(Wc2)
    bc2p = jnp.zeros((d,), jnp.float32).at[:Wc2.shape[1]].set(bc2)

    np_ = _pad_rows(n)
    degpart = _sc_degree(dst, n).reshape(NC, np_, DEG_W)
    h1s, dinv2 = _tc_prep(degpart, x, W1)

    acc1 = _sc_aggregate(h1s, src, dst).reshape(NC, np_, d)
    h2s = _tc_layer(acc1, h1s, dinv2, b1, g1, beta1, W2)

    acc2 = _sc_aggregate(h2s, src, dst).reshape(NC, np_, d)
    h3s = _tc_layer(acc2, h2s, dinv2, b2, g2, beta2, W3)

    acc3 = _sc_aggregate(h3s, src, dst).reshape(NC, np_, d)
    node_embeddings, logits_p, embedding = _tc_tail(
        acc3, h3s, dinv2, b3, g3, beta3, batch, Wc1p, bc1p, Wc2p, bc2p, We, be)

    return logits_p[:, :Wc2.shape[1]], embedding, node_embeddings


# speedup vs baseline: 26.3312x; 1.1613x over previous
import functools

import jax
import jax.numpy as jnp
from jax import lax
from jax.experimental import pallas as pl
from jax.experimental.pallas import tpu as pltpu
from jax.experimental.pallas import tpu_sc as plsc

NC = 2
NS = 16
LANES = 16
CHUNK = 64
DEG_W = 128
EPS = 1e-5

_MESH = plsc.VectorSubcoreMesh(core_axis_name="c", subcore_axis_name="s")
_PREC = lax.Precision.HIGHEST


def _dot(a, b):
    return jnp.dot(a, b, precision=_PREC, preferred_element_type=jnp.float32)


def _pad_rows(n):
    return -(-n // (8 * NS)) * (8 * NS)


def _sc_degree(dst, n):
    E = dst.shape[0]
    np_ = _pad_rows(n)
    ne_core = E // NC
    ne_w = ne_core // NS
    nfull = ne_w // CHUNK
    tail = ne_w % CHUNK
    rows_sub = np_ // NS
    w = DEG_W
    KB = 13

    @functools.partial(
        pl.kernel,
        out_type=jax.ShapeDtypeStruct((NC * np_, w), jnp.float32),
        mesh=_MESH,
        scratch_types=[
            pltpu.VMEM((nfull, CHUNK), jnp.int32),
            pltpu.VMEM((LANES,), jnp.int32),
            pltpu.VMEM((CHUNK, w), jnp.float32),
            pltpu.VMEM((CHUNK, w), jnp.float32),
            pltpu.VMEM_SHARED((np_, w), jnp.float32),
            pltpu.SemaphoreType.DMA,
            pltpu.SemaphoreType.DMA,
        ],
    )
    def deg_kernel(dst_hbm, out_hbm, dsti_all, idxt_v, ones_v, zeros_v, acc_sh,
                   isem, ssem):
        cid = lax.axis_index("c")
        sid = lax.axis_index("s")

        @pl.loop(0, CHUNK)
        def _(r):
            for c8 in range(w // LANES):
                ones_v[r, pl.ds(c8 * LANES, LANES)] = jnp.full((LANES,), 1.0, jnp.float32)
                zeros_v[r, pl.ds(c8 * LANES, LANES)] = jnp.zeros((LANES,), jnp.float32)

        base0 = cid * ne_core + sid * ne_w

        @pl.loop(0, nfull)
        def _(j):
            pltpu.async_copy(dst_hbm.at[pl.ds(base0 + j * CHUNK, CHUNK)],
                             dsti_all.at[j], isem)
        if tail:
            pltpu.sync_copy(dst_hbm.at[pl.ds(base0 + nfull * CHUNK, tail)], idxt_v)

        r0 = sid * rows_sub
        done = 0
        while done < rows_sub:
            step = min(CHUNK, rows_sub - done)
            pltpu.sync_copy(zeros_v.at[pl.ds(0, step)],
                            acc_sh.at[pl.ds(r0 + done, step)])
            done += step
        @pl.loop(0, nfull)
        def _(j):
            pltpu.make_async_copy(dst_hbm.at[pl.ds(base0 + j * CHUNK, CHUNK)],
                                  dsti_all.at[j], isem).wait()
        plsc.subcore_barrier()

        assert nfull % KB == 0
        @pl.loop(0, nfull, step=KB)
        def _(b0):
            for jo in range(KB):
                pltpu.async_copy(ones_v, acc_sh.at[dsti_all.at[b0 + jo]], ssem,
                                 add=True)
            for jo in range(KB):
                pltpu.make_async_copy(ones_v, acc_sh.at[dsti_all.at[b0 + jo]],
                                      ssem).wait()
        if tail:
            pltpu.sync_copy(ones_v.at[pl.ds(0, tail)], acc_sh.at[idxt_v], add=True)

        plsc.subcore_barrier()
        pltpu.sync_copy(acc_sh.at[pl.ds(r0, rows_sub)],
                        out_hbm.at[pl.ds(cid * np_ + r0, rows_sub)])

    return deg_kernel(dst)


def _sc_aggregate(h, src, dst):
    n, d = h.shape
    np_ = _pad_rows(n)
    E = src.shape[0]
    ne_core = E // NC
    ne_w = ne_core // NS
    rows_sub = np_ // NS
    NBUF = 3
    NPH = 2
    ne_h = ne_w // NPH
    nfh = ne_h // CHUNK
    tailh = ne_h % CHUNK
    assert ne_w % NPH == 0 and nfh % NBUF == 0 and tailh % 8 == 0

    @functools.partial(
        pl.kernel,
        out_type=jax.ShapeDtypeStruct((NC * np_, d), jnp.float32),
        mesh=_MESH,
        scratch_types=[
            pltpu.VMEM((ne_h,), jnp.int32),
            pltpu.VMEM((nfh, CHUNK), jnp.int32),
            pltpu.VMEM((max(tailh, 8),), jnp.int32),
            pltpu.VMEM((CHUNK, d), jnp.float32),
            pltpu.VMEM((CHUNK, d), jnp.float32),
            pltpu.VMEM((CHUNK, d), jnp.float32),
            pltpu.VMEM_SHARED((np_, d), jnp.float32),
            pltpu.SemaphoreType.DMA,
            pltpu.SemaphoreType.DMA,
            pltpu.SemaphoreType.DMA,
            pltpu.SemaphoreType.DMA,
        ],
    )
    def agg_kernel(h_hbm, src_hbm, dst_hbm, out_hbm,
                   srci_all, dsti_all, idxt_v, rows_a, rows_b, rows_c, acc_sh,
                   gsem_a, gsem_b, gsem_c, isem):
        cid = lax.axis_index("c")
        sid = lax.axis_index("s")
        base0 = cid * ne_core + sid * ne_w
        bufs = [rows_a, rows_b, rows_c]
        sems = [gsem_a, gsem_b, gsem_c]

        @pl.loop(0, CHUNK)
        def _(r):
            for c8 in range(d // LANES):
                rows_a[r, pl.ds(c8 * LANES, LANES)] = jnp.zeros((LANES,), jnp.float32)

        r0 = sid * rows_sub
        done = 0
        while done < rows_sub:
            step = min(CHUNK, rows_sub - done)
            pltpu.sync_copy(rows_a.at[pl.ds(0, step)],
                            acc_sh.at[pl.ds(r0 + done, step)])
            done += step
        plsc.subcore_barrier()

        def run_phase(pbase):
            @pl.loop(0, nfh)
            def _(j):
                pltpu.async_copy(dst_hbm.at[pl.ds(pbase + j * CHUNK, CHUNK)],
                                 dsti_all.at[j], isem)
            pltpu.sync_copy(src_hbm.at[pl.ds(pbase, ne_h)], srci_all)

            @pl.loop(0, nfh)
            def _(j):
                pltpu.make_async_copy(dst_hbm.at[pl.ds(pbase + j * CHUNK, CHUNK)],
                                      dsti_all.at[j], isem).wait()

            def gather(j, buf, sem):
                return pltpu.async_copy(
                    h_hbm.at[srci_all.at[pl.ds(j * CHUNK, CHUNK)]], buf, sem)

            def gather_wait(j, buf, sem):
                pltpu.make_async_copy(
                    h_hbm.at[srci_all.at[pl.ds(j * CHUNK, CHUNK)]], buf, sem).wait()

            def scatter(j, buf):
                pltpu.sync_copy(buf, acc_sh.at[dsti_all.at[j]], add=True)

            for k in range(NBUF):
                gather(k, bufs[k], sems[k])

            @pl.loop(0, nfh - NBUF, step=NBUF)
            def _(j):
                for k in range(NBUF):
                    gather_wait(j + k, bufs[k], sems[k])
                    scatter(j + k, bufs[k])
                    gather(j + NBUF + k, bufs[k], sems[k])

            jj = nfh - NBUF
            for k in range(NBUF):
                gather_wait(jj + k, bufs[k], sems[k])
                scatter(jj + k, bufs[k])

            if tailh:
                b = pbase + nfh * CHUNK
                pltpu.sync_copy(src_hbm.at[pl.ds(b, tailh)], idxt_v)
                pltpu.async_copy(h_hbm.at[idxt_v], rows_a.at[pl.ds(0, tailh)],
                                 gsem_a).wait()
                pltpu.sync_copy(dst_hbm.at[pl.ds(b, tailh)], idxt_v)
                pltpu.sync_copy(rows_a.at[pl.ds(0, tailh)], acc_sh.at[idxt_v],
                                add=True)

        for ph in range(NPH):
            run_phase(base0 + ph * ne_h)

        plsc.subcore_barrier()
        pltpu.sync_copy(acc_sh.at[pl.ds(r0, rows_sub)],
                        out_hbm.at[pl.ds(cid * np_ + r0, rows_sub)])

    return agg_kernel(h, src, dst)


def _tc_prep(degpart, x, W1):
    n, d = x.shape

    def body(degp_ref, x_ref, w_ref, h1s_ref, dinv2_ref):
        deg = degp_ref[0, :n, 0:1] + degp_ref[1, :n, 0:1] + 1.0
        dinv = lax.rsqrt(jnp.maximum(deg, 1e-12))
        dinv2 = jnp.broadcast_to(dinv, (n, d))
        dinv2_ref[...] = dinv2
        h1s_ref[...] = _dot(x_ref[...], w_ref[...]) * dinv2

    return pl.pallas_call(
        body,
        out_shape=[jax.ShapeDtypeStruct((n, d), jnp.float32),
                   jax.ShapeDtypeStruct((n, d), jnp.float32)],
    )(degpart, x, W1)


def _combine_bn_relu(acc_ref, hs_ref, dinv2_ref, b_ref, g_ref, bt_ref):
    n = hs_ref.shape[0]
    h = (acc_ref[0, :n] + acc_ref[1, :n] + hs_ref[...]) * dinv2_ref[...] + b_ref[...]
    mu = jnp.mean(h, axis=0)
    var = jnp.mean((h - mu) ** 2, axis=0)
    return jnp.maximum((h - mu) * lax.rsqrt(var + EPS) * g_ref[...] + bt_ref[...],
                       0.0)


def _tc_layer(accpart, hs, dinv2, b, g, bt, Wn):
    n, d = hs.shape

    def body(acc_ref, hs_ref, dinv2_ref, b_ref, g_ref, bt_ref, w_ref, out_ref):
        hn = _combine_bn_relu(acc_ref, hs_ref, dinv2_ref, b_ref, g_ref, bt_ref)
        out_ref[...] = _dot(hn, w_ref[...]) * dinv2_ref[...]

    return pl.pallas_call(
        body,
        out_shape=jax.ShapeDtypeStruct((n, d), jnp.float32),
    )(accpart, hs, dinv2, b, g, bt, Wn)


def _tc_tail(accpart, hs, dinv2, b, g, bt, batch, Wc1p, bc1p, Wc2p, bc2p, We, be):
    n, d = hs.shape
    ngr = 16

    def body(acc_ref, hs_ref, dinv2_ref, b_ref, g_ref, bt_ref, batch_ref,
             wc1_ref, bc1_ref, wc2_ref, bc2_ref, we_ref, be_ref,
             ne_ref, logits_ref, emb_ref):
        hn = _combine_bn_relu(acc_ref, hs_ref, dinv2_ref, b_ref, g_ref, bt_ref)
        ne_ref[...] = hn
        gid = lax.broadcasted_iota(jnp.int32, (ngr, n), 0)
        mask = (batch_ref[...][None, :] == gid).astype(jnp.float32)
        cnt = jnp.maximum(jnp.sum(mask, axis=1), 1.0)
        pooled = _dot(mask, hn) / cnt[:, None]
        z = jnp.maximum(_dot(pooled, wc1_ref[...]) + bc1_ref[...], 0.0)
        logits_ref[...] = _dot(z, wc2_ref[...]) + bc2_ref[...]
        emb_ref[...] = _dot(pooled, we_ref[...]) + be_ref[...]

    return pl.pallas_call(
        body,
        out_shape=[jax.ShapeDtypeStruct((n, d), jnp.float32),
                   jax.ShapeDtypeStruct((ngr, d), jnp.float32),
                   jax.ShapeDtypeStruct((ngr, d), jnp.float32)],
    )(accpart, hs, dinv2, b, g, bt, batch, Wc1p, bc1p, Wc2p, bc2p, We, be)


def kernel(x, edge_index, batch, W1, b1, W2, b2, W3, b3, g1, beta1, g2, beta2,
           g3, beta3, Wc1, bc1, Wc2, bc2, We, be):
    n, d = x.shape
    src = edge_index[0]
    dst = edge_index[1]

    h1 = Wc1.shape[1]
    Wc1p = jnp.zeros((d, d), jnp.float32).at[:, :h1].set(Wc1)
    bc1p = jnp.zeros((d,), jnp.float32).at[:h1].set(bc1)
    Wc2p = jnp.zeros((d, d), jnp.float32).at[:h1, :Wc2.shape[1]].set(Wc2)
    bc2p = jnp.zeros((d,), jnp.float32).at[:Wc2.shape[1]].set(bc2)

    np_ = _pad_rows(n)
    degpart = _sc_degree(dst, n).reshape(NC, np_, DEG_W)
    h1s, dinv2 = _tc_prep(degpart, x, W1)

    acc1 = _sc_aggregate(h1s, src, dst).reshape(NC, np_, d)
    h2s = _tc_layer(acc1, h1s, dinv2, b1, g1, beta1, W2)

    acc2 = _sc_aggregate(h2s, src, dst).reshape(NC, np_, d)
    h3s = _tc_layer(acc2, h2s, dinv2, b2, g2, beta2, W3)

    acc3 = _sc_aggregate(h3s, src, dst).reshape(NC, np_, d)
    node_embeddings, logits_p, embedding = _tc_tail(
        acc3, h3s, dinv2, b3, g3, beta3, batch, Wc1p, bc1p, Wc2p, bc2p, We, be)

    return logits_p[:, :Wc2.shape[1]], embedding, node_embeddings


# speedup vs baseline: 27.0787x; 1.0284x over previous
import functools

import jax
import jax.numpy as jnp
from jax import lax
from jax.experimental import pallas as pl
from jax.experimental.pallas import tpu as pltpu
from jax.experimental.pallas import tpu_sc as plsc

NC = 2
NS = 16
LANES = 16
CHUNK = 32
DEG_W = 128
EPS = 1e-5

_MESH = plsc.VectorSubcoreMesh(core_axis_name="c", subcore_axis_name="s")
_PREC = lax.Precision.HIGHEST


def _dot(a, b):
    return jnp.dot(a, b, precision=_PREC, preferred_element_type=jnp.float32)


def _pad_rows(n):
    return -(-n // (8 * NS)) * (8 * NS)


def _sc_degree(dst, n):
    E = dst.shape[0]
    np_ = _pad_rows(n)
    ne_core = E // NC
    ne_w = ne_core // NS
    nfull = ne_w // CHUNK
    tail = ne_w % CHUNK
    rows_sub = np_ // NS
    w = DEG_W
    KB = 13

    @functools.partial(
        pl.kernel,
        out_type=jax.ShapeDtypeStruct((NC * np_, w), jnp.float32),
        mesh=_MESH,
        scratch_types=[
            pltpu.VMEM((nfull, CHUNK), jnp.int32),
            pltpu.VMEM((LANES,), jnp.int32),
            pltpu.VMEM((CHUNK, w), jnp.float32),
            pltpu.VMEM((CHUNK, w), jnp.float32),
            pltpu.VMEM_SHARED((np_, w), jnp.float32),
            pltpu.SemaphoreType.DMA,
            pltpu.SemaphoreType.DMA,
        ],
    )
    def deg_kernel(dst_hbm, out_hbm, dsti_all, idxt_v, ones_v, zeros_v, acc_sh,
                   isem, ssem):
        cid = lax.axis_index("c")
        sid = lax.axis_index("s")

        @pl.loop(0, CHUNK)
        def _(r):
            for c8 in range(w // LANES):
                ones_v[r, pl.ds(c8 * LANES, LANES)] = jnp.full((LANES,), 1.0, jnp.float32)
                zeros_v[r, pl.ds(c8 * LANES, LANES)] = jnp.zeros((LANES,), jnp.float32)

        base0 = cid * ne_core + sid * ne_w

        @pl.loop(0, nfull)
        def _(j):
            pltpu.async_copy(dst_hbm.at[pl.ds(base0 + j * CHUNK, CHUNK)],
                             dsti_all.at[j], isem)
        if tail:
            pltpu.sync_copy(dst_hbm.at[pl.ds(base0 + nfull * CHUNK, tail)], idxt_v)

        r0 = sid * rows_sub
        done = 0
        while done < rows_sub:
            step = min(CHUNK, rows_sub - done)
            pltpu.sync_copy(zeros_v.at[pl.ds(0, step)],
                            acc_sh.at[pl.ds(r0 + done, step)])
            done += step
        @pl.loop(0, nfull)
        def _(j):
            pltpu.make_async_copy(dst_hbm.at[pl.ds(base0 + j * CHUNK, CHUNK)],
                                  dsti_all.at[j], isem).wait()
        plsc.subcore_barrier()

        assert nfull % KB == 0
        @pl.loop(0, nfull, step=KB)
        def _(b0):
            for jo in range(KB):
                pltpu.async_copy(ones_v, acc_sh.at[dsti_all.at[b0 + jo]], ssem,
                                 add=True)
            for jo in range(KB):
                pltpu.make_async_copy(ones_v, acc_sh.at[dsti_all.at[b0 + jo]],
                                      ssem).wait()
        if tail:
            pltpu.sync_copy(ones_v.at[pl.ds(0, tail)], acc_sh.at[idxt_v], add=True)

        plsc.subcore_barrier()
        pltpu.sync_copy(acc_sh.at[pl.ds(r0, rows_sub)],
                        out_hbm.at[pl.ds(cid * np_ + r0, rows_sub)])

    return deg_kernel(dst)


def _sc_aggregate(h, src, dst):
    n, d = h.shape
    np_ = _pad_rows(n)
    E = src.shape[0]
    ne_core = E // NC
    ne_w = ne_core // NS
    rows_sub = np_ // NS
    NBUF = 6
    NPH = 2
    ne_h = ne_w // NPH
    nfh = ne_h // CHUNK
    tailh = ne_h % CHUNK
    assert ne_w % NPH == 0 and nfh % NBUF == 0 and tailh % 8 == 0

    @functools.partial(
        pl.kernel,
        out_type=jax.ShapeDtypeStruct((NC * np_, d), jnp.float32),
        mesh=_MESH,
        scratch_types=[
            pltpu.VMEM((ne_h,), jnp.int32),
            pltpu.VMEM((nfh, CHUNK), jnp.int32),
            pltpu.VMEM((max(tailh, 8),), jnp.int32),
        ] + [pltpu.VMEM((CHUNK, d), jnp.float32)] * NBUF + [
            pltpu.VMEM_SHARED((n, d), jnp.float32),
        ] + [pltpu.SemaphoreType.DMA] * (NBUF + 1),
    )
    def agg_kernel(h_hbm, src_hbm, dst_hbm, out_hbm,
                   srci_all, dsti_all, idxt_v, *rest):
        bufs = list(rest[:NBUF])
        acc_sh = rest[NBUF]
        sems = list(rest[NBUF + 1:2 * NBUF + 1])
        isem = rest[2 * NBUF + 1]
        rows_a = bufs[0]
        gsem_a = sems[0]
        cid = lax.axis_index("c")
        sid = lax.axis_index("s")
        base0 = cid * ne_core + sid * ne_w

        @pl.loop(0, CHUNK)
        def _(r):
            for c8 in range(d // LANES):
                rows_a[r, pl.ds(c8 * LANES, LANES)] = jnp.zeros((LANES,), jnp.float32)

        r0 = sid * rows_sub
        rows_last = n - (NS - 1) * rows_sub

        def zero_rows(count):
            done = 0
            while done < count:
                step = min(CHUNK, count - done)
                pltpu.sync_copy(rows_a.at[pl.ds(0, step)],
                                acc_sh.at[pl.ds(r0 + done, step)])
                done += step

        @pl.when(sid < NS - 1)
        def _():
            zero_rows(rows_sub)

        @pl.when(sid == NS - 1)
        def _():
            zero_rows(rows_last)
        plsc.subcore_barrier()

        def run_phase(pbase):
            @pl.loop(0, nfh)
            def _(j):
                pltpu.async_copy(dst_hbm.at[pl.ds(pbase + j * CHUNK, CHUNK)],
                                 dsti_all.at[j], isem)
            pltpu.sync_copy(src_hbm.at[pl.ds(pbase, ne_h)], srci_all)

            @pl.loop(0, nfh)
            def _(j):
                pltpu.make_async_copy(dst_hbm.at[pl.ds(pbase + j * CHUNK, CHUNK)],
                                      dsti_all.at[j], isem).wait()

            def gather(j, buf, sem):
                return pltpu.async_copy(
                    h_hbm.at[srci_all.at[pl.ds(j * CHUNK, CHUNK)]], buf, sem)

            def gather_wait(j, buf, sem):
                pltpu.make_async_copy(
                    h_hbm.at[srci_all.at[pl.ds(j * CHUNK, CHUNK)]], buf, sem).wait()

            def scatter(j, buf):
                pltpu.sync_copy(buf, acc_sh.at[dsti_all.at[j]], add=True)

            for k in range(NBUF):
                gather(k, bufs[k], sems[k])

            @pl.loop(0, nfh - NBUF, step=NBUF)
            def _(j):
                for k in range(NBUF):
                    gather_wait(j + k, bufs[k], sems[k])
                    scatter(j + k, bufs[k])
                    gather(j + NBUF + k, bufs[k], sems[k])

            jj = nfh - NBUF
            for k in range(NBUF):
                gather_wait(jj + k, bufs[k], sems[k])
                scatter(jj + k, bufs[k])

            if tailh:
                b = pbase + nfh * CHUNK
                pltpu.sync_copy(src_hbm.at[pl.ds(b, tailh)], idxt_v)
                pltpu.async_copy(h_hbm.at[idxt_v], rows_a.at[pl.ds(0, tailh)],
                                 gsem_a).wait()
                pltpu.sync_copy(dst_hbm.at[pl.ds(b, tailh)], idxt_v)
                pltpu.sync_copy(rows_a.at[pl.ds(0, tailh)], acc_sh.at[idxt_v],
                                add=True)

        for ph in range(NPH):
            run_phase(base0 + ph * ne_h)

        plsc.subcore_barrier()

        @pl.when(sid < NS - 1)
        def _():
            pltpu.sync_copy(acc_sh.at[pl.ds(r0, rows_sub)],
                            out_hbm.at[pl.ds(cid * np_ + r0, rows_sub)])

        @pl.when(sid == NS - 1)
        def _():
            pltpu.sync_copy(acc_sh.at[pl.ds(r0, rows_last)],
                            out_hbm.at[pl.ds(cid * np_ + r0, rows_last)])

    return agg_kernel(h, src, dst)


def _tc_prep(degpart, x, W1):
    n, d = x.shape

    def body(degp_ref, x_ref, w_ref, h1s_ref, dinv2_ref):
        deg = degp_ref[0, :n, 0:1] + degp_ref[1, :n, 0:1] + 1.0
        dinv = lax.rsqrt(jnp.maximum(deg, 1e-12))
        dinv2 = jnp.broadcast_to(dinv, (n, d))
        dinv2_ref[...] = dinv2
        h1s_ref[...] = _dot(x_ref[...], w_ref[...]) * dinv2

    return pl.pallas_call(
        body,
        out_shape=[jax.ShapeDtypeStruct((n, d), jnp.float32),
                   jax.ShapeDtypeStruct((n, d), jnp.float32)],
    )(degpart, x, W1)


def _combine_bn_relu(acc_ref, hs_ref, dinv2_ref, b_ref, g_ref, bt_ref):
    n = hs_ref.shape[0]
    h = (acc_ref[0, :n] + acc_ref[1, :n] + hs_ref[...]) * dinv2_ref[...] + b_ref[...]
    mu = jnp.mean(h, axis=0)
    var = jnp.mean((h - mu) ** 2, axis=0)
    return jnp.maximum((h - mu) * lax.rsqrt(var + EPS) * g_ref[...] + bt_ref[...],
                       0.0)


def _tc_layer(accpart, hs, dinv2, b, g, bt, Wn):
    n, d = hs.shape

    def body(acc_ref, hs_ref, dinv2_ref, b_ref, g_ref, bt_ref, w_ref, out_ref):
        hn = _combine_bn_relu(acc_ref, hs_ref, dinv2_ref, b_ref, g_ref, bt_ref)
        out_ref[...] = _dot(hn, w_ref[...]) * dinv2_ref[...]

    return pl.pallas_call(
        body,
        out_shape=jax.ShapeDtypeStruct((n, d), jnp.float32),
    )(accpart, hs, dinv2, b, g, bt, Wn)


def _tc_tail(accpart, hs, dinv2, b, g, bt, batch, Wc1p, bc1p, Wc2p, bc2p, We, be):
    n, d = hs.shape
    ngr = 16

    def body(acc_ref, hs_ref, dinv2_ref, b_ref, g_ref, bt_ref, batch_ref,
             wc1_ref, bc1_ref, wc2_ref, bc2_ref, we_ref, be_ref,
             ne_ref, logits_ref, emb_ref):
        hn = _combine_bn_relu(acc_ref, hs_ref, dinv2_ref, b_ref, g_ref, bt_ref)
        ne_ref[...] = hn
        gid = lax.broadcasted_iota(jnp.int32, (ngr, n), 0)
        mask = (batch_ref[...][None, :] == gid).astype(jnp.float32)
        cnt = jnp.maximum(jnp.sum(mask, axis=1), 1.0)
        pooled = _dot(mask, hn) / cnt[:, None]
        z = jnp.maximum(_dot(pooled, wc1_ref[...]) + bc1_ref[...], 0.0)
        logits_ref[...] = _dot(z, wc2_ref[...]) + bc2_ref[...]
        emb_ref[...] = _dot(pooled, we_ref[...]) + be_ref[...]

    return pl.pallas_call(
        body,
        out_shape=[jax.ShapeDtypeStruct((n, d), jnp.float32),
                   jax.ShapeDtypeStruct((ngr, d), jnp.float32),
                   jax.ShapeDtypeStruct((ngr, d), jnp.float32)],
    )(accpart, hs, dinv2, b, g, bt, batch, Wc1p, bc1p, Wc2p, bc2p, We, be)


def kernel(x, edge_index, batch, W1, b1, W2, b2, W3, b3, g1, beta1, g2, beta2,
           g3, beta3, Wc1, bc1, Wc2, bc2, We, be):
    n, d = x.shape
    src = edge_index[0]
    dst = edge_index[1]

    h1 = Wc1.shape[1]
    Wc1p = jnp.zeros((d, d), jnp.float32).at[:, :h1].set(Wc1)
    bc1p = jnp.zeros((d,), jnp.float32).at[:h1].set(bc1)
    Wc2p = jnp.zeros((d, d), jnp.float32).at[:h1, :Wc2.shape[1]].set(Wc2)
    bc2p = jnp.zeros((d,), jnp.float32).at[:Wc2.shape[1]].set(bc2)

    np_ = _pad_rows(n)
    degpart = _sc_degree(dst, n).reshape(NC, np_, DEG_W)
    h1s, dinv2 = _tc_prep(degpart, x, W1)

    acc1 = _sc_aggregate(h1s, src, dst).reshape(NC, np_, d)
    h2s = _tc_layer(acc1, h1s, dinv2, b1, g1, beta1, W2)

    acc2 = _sc_aggregate(h2s, src, dst).reshape(NC, np_, d)
    h3s = _tc_layer(acc2, h2s, dinv2, b2, g2, beta2, W3)

    acc3 = _sc_aggregate(h3s, src, dst).reshape(NC, np_, d)
    node_embeddings, logits_p, embedding = _tc_tail(
        acc3, h3s, dinv2, b3, g3, beta3, batch, Wc1p, bc1p, Wc2p, bc2p, We, be)

    return logits_p[:, :Wc2.shape[1]], embedding, node_embeddings


# speedup vs baseline: 27.0996x; 1.0008x over previous
import functools

import jax
import jax.numpy as jnp
from jax import lax
from jax.experimental import pallas as pl
from jax.experimental.pallas import tpu as pltpu
from jax.experimental.pallas import tpu_sc as plsc

NC = 2
NS = 16
LANES = 16
CHUNK = 32
DEG_W = 128
EPS = 1e-5

_MESH = plsc.VectorSubcoreMesh(core_axis_name="c", subcore_axis_name="s")
_PREC = lax.Precision.HIGHEST


def _dot(a, b):
    return jnp.dot(a, b, precision=_PREC, preferred_element_type=jnp.float32)


def _pad_rows(n):
    return -(-n // (8 * NS)) * (8 * NS)


def _sc_degree(dst, n):
    E = dst.shape[0]
    np_ = _pad_rows(n)
    ne_core = E // NC
    ne_w = ne_core // NS
    nfull = ne_w // CHUNK
    tail = ne_w % CHUNK
    rows_sub = np_ // NS
    w = DEG_W
    KB = 13

    @functools.partial(
        pl.kernel,
        out_type=jax.ShapeDtypeStruct((NC * np_, w), jnp.float32),
        mesh=_MESH,
        scratch_types=[
            pltpu.VMEM((nfull, CHUNK), jnp.int32),
            pltpu.VMEM((LANES,), jnp.int32),
            pltpu.VMEM((CHUNK, w), jnp.float32),
            pltpu.VMEM((CHUNK, w), jnp.float32),
            pltpu.VMEM_SHARED((np_, w), jnp.float32),
            pltpu.SemaphoreType.DMA,
            pltpu.SemaphoreType.DMA,
        ],
    )
    def deg_kernel(dst_hbm, out_hbm, dsti_all, idxt_v, ones_v, zeros_v, acc_sh,
                   isem, ssem):
        cid = lax.axis_index("c")
        sid = lax.axis_index("s")

        @pl.loop(0, CHUNK)
        def _(r):
            for c8 in range(w // LANES):
                ones_v[r, pl.ds(c8 * LANES, LANES)] = jnp.full((LANES,), 1.0, jnp.float32)
                zeros_v[r, pl.ds(c8 * LANES, LANES)] = jnp.zeros((LANES,), jnp.float32)

        base0 = cid * ne_core + sid * ne_w

        @pl.loop(0, nfull)
        def _(j):
            pltpu.async_copy(dst_hbm.at[pl.ds(base0 + j * CHUNK, CHUNK)],
                             dsti_all.at[j], isem)
        if tail:
            pltpu.sync_copy(dst_hbm.at[pl.ds(base0 + nfull * CHUNK, tail)], idxt_v)

        r0 = sid * rows_sub
        done = 0
        while done < rows_sub:
            step = min(CHUNK, rows_sub - done)
            pltpu.sync_copy(zeros_v.at[pl.ds(0, step)],
                            acc_sh.at[pl.ds(r0 + done, step)])
            done += step
        @pl.loop(0, nfull)
        def _(j):
            pltpu.make_async_copy(dst_hbm.at[pl.ds(base0 + j * CHUNK, CHUNK)],
                                  dsti_all.at[j], isem).wait()
        plsc.subcore_barrier()

        assert nfull % KB == 0
        @pl.loop(0, nfull, step=KB)
        def _(b0):
            for jo in range(KB):
                pltpu.async_copy(ones_v, acc_sh.at[dsti_all.at[b0 + jo]], ssem,
                                 add=True)
            for jo in range(KB):
                pltpu.make_async_copy(ones_v, acc_sh.at[dsti_all.at[b0 + jo]],
                                      ssem).wait()
        if tail:
            pltpu.sync_copy(ones_v.at[pl.ds(0, tail)], acc_sh.at[idxt_v], add=True)

        plsc.subcore_barrier()
        pltpu.sync_copy(acc_sh.at[pl.ds(r0, rows_sub)],
                        out_hbm.at[pl.ds(cid * np_ + r0, rows_sub)])

    return deg_kernel(dst)


def _sc_aggregate(h, src, dst):
    n, d = h.shape
    np_ = _pad_rows(n)
    E = src.shape[0]
    ne_core = E // NC
    ne_w = ne_core // NS
    rows_sub = np_ // NS
    NBUF = 6
    NPH = 2
    ne_h = ne_w // NPH
    nfh = ne_h // CHUNK
    tailh = ne_h % CHUNK
    assert ne_w % NPH == 0 and nfh % NBUF == 0 and tailh % 8 == 0

    @functools.partial(
        pl.kernel,
        out_type=jax.ShapeDtypeStruct((NC * np_, d), jnp.float32),
        mesh=_MESH,
        scratch_types=[
            pltpu.VMEM((ne_h,), jnp.int32),
            pltpu.VMEM((nfh, CHUNK), jnp.int32),
            pltpu.VMEM((max(tailh, 8),), jnp.int32),
        ] + [pltpu.VMEM((CHUNK, d), jnp.float32)] * NBUF + [
            pltpu.VMEM_SHARED((n, d), jnp.float32),
        ] + [pltpu.SemaphoreType.DMA] * (NBUF + 1),
    )
    def agg_kernel(h_hbm, src_hbm, dst_hbm, out_hbm,
                   srci_all, dsti_all, idxt_v, *rest):
        bufs = list(rest[:NBUF])
        acc_sh = rest[NBUF]
        sems = list(rest[NBUF + 1:2 * NBUF + 1])
        isem = rest[2 * NBUF + 1]
        rows_a = bufs[0]
        gsem_a = sems[0]
        cid = lax.axis_index("c")
        sid = lax.axis_index("s")
        base0 = cid * ne_core + sid * ne_w

        @pl.loop(0, CHUNK)
        def _(r):
            for c8 in range(d // LANES):
                rows_a[r, pl.ds(c8 * LANES, LANES)] = jnp.zeros((LANES,), jnp.float32)

        r0 = sid * rows_sub
        rows_last = n - (NS - 1) * rows_sub

        def zero_rows(count):
            done = 0
            while done < count:
                step = min(CHUNK, count - done)
                pltpu.sync_copy(rows_a.at[pl.ds(0, step)],
                                acc_sh.at[pl.ds(r0 + done, step)])
                done += step

        @pl.when(sid < NS - 1)
        def _():
            zero_rows(rows_sub)

        @pl.when(sid == NS - 1)
        def _():
            zero_rows(rows_last)
        plsc.subcore_barrier()

        def run_phase(pbase):
            @pl.loop(0, nfh)
            def _(j):
                pltpu.async_copy(dst_hbm.at[pl.ds(pbase + j * CHUNK, CHUNK)],
                                 dsti_all.at[j], isem)
            pltpu.sync_copy(src_hbm.at[pl.ds(pbase, ne_h)], srci_all)

            @pl.loop(0, nfh)
            def _(j):
                pltpu.make_async_copy(dst_hbm.at[pl.ds(pbase + j * CHUNK, CHUNK)],
                                      dsti_all.at[j], isem).wait()

            def gather(j, buf, sem):
                return pltpu.async_copy(
                    h_hbm.at[srci_all.at[pl.ds(j * CHUNK, CHUNK)]], buf, sem)

            def gather_wait(j, buf, sem):
                pltpu.make_async_copy(
                    h_hbm.at[srci_all.at[pl.ds(j * CHUNK, CHUNK)]], buf, sem).wait()

            def scatter(j, buf):
                pltpu.sync_copy(buf, acc_sh.at[dsti_all.at[j]], add=True)

            for k in range(NBUF):
                gather(k, bufs[k], sems[k])

            @pl.loop(0, nfh - NBUF, step=NBUF)
            def _(j):
                for k in range(NBUF):
                    gather_wait(j + k, bufs[k], sems[k])
                    scatter(j + k, bufs[k])
                    gather(j + NBUF + k, bufs[k], sems[k])

            jj = nfh - NBUF
            for k in range(NBUF):
                gather_wait(jj + k, bufs[k], sems[k])
                scatter(jj + k, bufs[k])

            if tailh:
                b = pbase + nfh * CHUNK
                pltpu.sync_copy(src_hbm.at[pl.ds(b, tailh)], idxt_v)
                pltpu.async_copy(h_hbm.at[idxt_v], rows_a.at[pl.ds(0, tailh)],
                                 gsem_a).wait()
                pltpu.sync_copy(dst_hbm.at[pl.ds(b, tailh)], idxt_v)
                pltpu.sync_copy(rows_a.at[pl.ds(0, tailh)], acc_sh.at[idxt_v],
                                add=True)

        for ph in range(NPH):
            run_phase(base0 + ph * ne_h)

        plsc.subcore_barrier()

        @pl.when(sid < NS - 1)
        def _():
            pltpu.sync_copy(acc_sh.at[pl.ds(r0, rows_sub)],
                            out_hbm.at[pl.ds(cid * np_ + r0, rows_sub)])

        @pl.when(sid == NS - 1)
        def _():
            pltpu.sync_copy(acc_sh.at[pl.ds(r0, rows_last)],
                            out_hbm.at[pl.ds(cid * np_ + r0, rows_last)])

    return agg_kernel(h, src, dst)


def _tc_mm1(x, W1):
    n, d = x.shape

    def body(x_ref, w_ref, h1_ref):
        h1_ref[...] = _dot(x_ref[...], w_ref[...])

    return pl.pallas_call(
        body,
        out_shape=jax.ShapeDtypeStruct((n, d), jnp.float32),
    )(x, W1)


def _tc_scale(degpart, h1):
    n, d = h1.shape

    def body(degp_ref, h1_ref, h1s_ref, dinv2_ref):
        deg = degp_ref[0, :n, 0:1] + degp_ref[1, :n, 0:1] + 1.0
        dinv = lax.rsqrt(jnp.maximum(deg, 1e-12))
        dinv2 = jnp.broadcast_to(dinv, (n, d))
        dinv2_ref[...] = dinv2
        h1s_ref[...] = h1_ref[...] * dinv2

    return pl.pallas_call(
        body,
        out_shape=[jax.ShapeDtypeStruct((n, d), jnp.float32),
                   jax.ShapeDtypeStruct((n, d), jnp.float32)],
    )(degpart, h1)


def _combine_bn_relu(acc_ref, hs_ref, dinv2_ref, b_ref, g_ref, bt_ref):
    n = hs_ref.shape[0]
    h = (acc_ref[0, :n] + acc_ref[1, :n] + hs_ref[...]) * dinv2_ref[...] + b_ref[...]
    mu = jnp.mean(h, axis=0)
    var = jnp.mean((h - mu) ** 2, axis=0)
    return jnp.maximum((h - mu) * lax.rsqrt(var + EPS) * g_ref[...] + bt_ref[...],
                       0.0)


def _tc_layer(accpart, hs, dinv2, b, g, bt, Wn):
    n, d = hs.shape

    def body(acc_ref, hs_ref, dinv2_ref, b_ref, g_ref, bt_ref, w_ref, out_ref):
        hn = _combine_bn_relu(acc_ref, hs_ref, dinv2_ref, b_ref, g_ref, bt_ref)
        out_ref[...] = _dot(hn, w_ref[...]) * dinv2_ref[...]

    return pl.pallas_call(
        body,
        out_shape=jax.ShapeDtypeStruct((n, d), jnp.float32),
    )(accpart, hs, dinv2, b, g, bt, Wn)


def _tc_tail(accpart, hs, dinv2, b, g, bt, batch, Wc1p, bc1p, Wc2p, bc2p, We, be):
    n, d = hs.shape
    ngr = 16

    def body(acc_ref, hs_ref, dinv2_ref, b_ref, g_ref, bt_ref, batch_ref,
             wc1_ref, bc1_ref, wc2_ref, bc2_ref, we_ref, be_ref,
             ne_ref, logits_ref, emb_ref):
        hn = _combine_bn_relu(acc_ref, hs_ref, dinv2_ref, b_ref, g_ref, bt_ref)
        ne_ref[...] = hn
        gid = lax.broadcasted_iota(jnp.int32, (ngr, n), 0)
        mask = (batch_ref[...][None, :] == gid).astype(jnp.float32)
        cnt = jnp.maximum(jnp.sum(mask, axis=1), 1.0)
        pooled = _dot(mask, hn) / cnt[:, None]
        z = jnp.maximum(_dot(pooled, wc1_ref[...]) + bc1_ref[...], 0.0)
        logits_ref[...] = _dot(z, wc2_ref[...]) + bc2_ref[...]
        emb_ref[...] = _dot(pooled, we_ref[...]) + be_ref[...]

    return pl.pallas_call(
        body,
        out_shape=[jax.ShapeDtypeStruct((n, d), jnp.float32),
                   jax.ShapeDtypeStruct((ngr, d), jnp.float32),
                   jax.ShapeDtypeStruct((ngr, d), jnp.float32)],
    )(accpart, hs, dinv2, b, g, bt, batch, Wc1p, bc1p, Wc2p, bc2p, We, be)


def kernel(x, edge_index, batch, W1, b1, W2, b2, W3, b3, g1, beta1, g2, beta2,
           g3, beta3, Wc1, bc1, Wc2, bc2, We, be):
    n, d = x.shape
    src = edge_index[0]
    dst = edge_index[1]

    h1 = Wc1.shape[1]
    Wc1p = jnp.zeros((d, d), jnp.float32).at[:, :h1].set(Wc1)
    bc1p = jnp.zeros((d,), jnp.float32).at[:h1].set(bc1)
    Wc2p = jnp.zeros((d, d), jnp.float32).at[:h1, :Wc2.shape[1]].set(Wc2)
    bc2p = jnp.zeros((d,), jnp.float32).at[:Wc2.shape[1]].set(bc2)

    np_ = _pad_rows(n)
    degpart = _sc_degree(dst, n).reshape(NC, np_, DEG_W)
    h1 = _tc_mm1(x, W1)
    h1s, dinv2 = _tc_scale(degpart, h1)

    acc1 = _sc_aggregate(h1s, src, dst).reshape(NC, np_, d)
    h2s = _tc_layer(acc1, h1s, dinv2, b1, g1, beta1, W2)

    acc2 = _sc_aggregate(h2s, src, dst).reshape(NC, np_, d)
    h3s = _tc_layer(acc2, h2s, dinv2, b2, g2, beta2, W3)

    acc3 = _sc_aggregate(h3s, src, dst).reshape(NC, np_, d)
    node_embeddings, logits_p, embedding = _tc_tail(
        acc3, h3s, dinv2, b3, g3, beta3, batch, Wc1p, bc1p, Wc2p, bc2p, We, be)

    return logits_p[:, :Wc2.shape[1]], embedding, node_embeddings
